# baseline (speedup 1.0000x reference)
import jax
import jax.numpy as jnp
from jax import lax
from jax.experimental import pallas as pl
from jax.experimental.pallas import tpu as pltpu

_LAYERS = 8
_KW = 5
_HW_IN = 64
_HW_FC = 32
_CLASSES = 2
_SLOPE = 0.01
_BS = 64
_BT = 2 * _BS

_OUT_SIZES = [_HW_IN - layer * (_KW - 1) - (_KW - 1) for layer in range(_LAYERS)]
_MXU_ROWS = [(out + 1) // 2 for out in _OUT_SIZES]


def _convnet_body(x_ref, cw_ref, cb_ref, s_ref, fwp_ref, out_ref):
    half_a = x_ref[0:_BS, :, :]
    half_b = x_ref[_BS:2 * _BS, :, :]
    packed = jnp.concatenate([half_a, half_b], axis=2)
    act = jnp.transpose(packed, (1, 0, 2))
    size = _HW_IN
    for layer in range(_LAYERS):
        out_size = size - (_KW - 1)
        r = _MXU_ROWS[layer]
        pieces = []
        if r > 0:
            y = None
            for di in range(_KW):
                lhs = act[di:di + r].reshape(r * _BS, _BT)
                t = lax.dot_general(
                    lhs, s_ref[layer, di],
                    (((1,), (0,)), ((), ())),
                    preferred_element_type=jnp.float32,
                )
                y = t if y is None else y + t
            pieces.append(y.reshape(r, _BS, _BT))
        if r < out_size:
            parts = []
            for dj in range(_KW):
                shv = act if dj == 0 else pltpu.roll(act, _BT - dj, 2)
                p = None
                for di in range(_KW):
                    w = cw_ref[layer * _KW * _KW + di * _KW + dj]
                    term = shv[r + di:di + out_size] * w
                    p = term if p is None else p + term
                parts.append(p)
            pieces.append(
                ((parts[0] + parts[1]) + (parts[2] + parts[3])) + parts[4])
        acc = pieces[0] if len(pieces) == 1 else jnp.concatenate(pieces, axis=0)
        acc = acc + cb_ref[layer]
        act = jnp.maximum(acc, _SLOPE * acc)
        size = out_size
    for c in range(_CLASSES):
        p = act * fwp_ref[c]
        s = jnp.sum(p, axis=0)
        for k in (32, 16, 8, 4, 2, 1):
            s = s + pltpu.roll(s, _BT - k, 1)
        out_ref[c, :, :] = s


def kernel(x, conv_w, conv_b, fc_w, fc_b):
    n = x.shape[0]
    n_pad = ((n + _BT - 1) // _BT) * _BT
    tiles = n_pad // _BT

    xs = x[:, 0, :, :].astype(jnp.float32)
    if n_pad != n:
        xs = jnp.pad(xs, ((0, n_pad - n), (0, 0), (0, 0)))

    cw = conv_w.reshape(_LAYERS * _KW * _KW).astype(jnp.float32)
    cb = conv_b.reshape(_LAYERS).astype(jnp.float32)
    ar = jnp.arange(_BT)
    kk, jj = ar[:, None], ar[None, :]
    same = (kk // _HW_IN) == (jj // _HW_IN)
    wl = conv_w.reshape(_LAYERS, _KW, _KW).astype(jnp.float32)
    smat = jnp.zeros((_LAYERS, _KW, _BT, _BT), jnp.float32)
    for dj in range(_KW):
        mask = ((kk - jj) == dj) & same
        smat = smat + jnp.where(mask, 1.0, 0.0) * wl[:, :, dj][:, :, None, None]
    fw = fc_w.reshape(_CLASSES, _HW_FC, _HW_FC).astype(jnp.float32)
    fw = jnp.pad(fw, ((0, 0), (0, 0), (0, _HW_IN - _HW_FC)))
    fwp = jnp.concatenate([fw, fw], axis=-1).reshape(_CLASSES, _HW_FC, 1, _BT)

    out = pl.pallas_call(
        _convnet_body,
        out_shape=jax.ShapeDtypeStruct((_CLASSES, n_pad // 2, _BT), jnp.float32),
        grid=(tiles,),
        in_specs=[
            pl.BlockSpec((_BT, _HW_IN, _HW_IN), lambda i: (i, 0, 0)),
            pl.BlockSpec(memory_space=pltpu.MemorySpace.SMEM),
            pl.BlockSpec(memory_space=pltpu.MemorySpace.SMEM),
            pl.BlockSpec(memory_space=pltpu.MemorySpace.VMEM),
            pl.BlockSpec(memory_space=pltpu.MemorySpace.VMEM),
        ],
        out_specs=pl.BlockSpec((_CLASSES, _BS, _BT), lambda i: (0, i, 0)),
        compiler_params=pltpu.CompilerParams(
            dimension_semantics=("parallel",),
            vmem_limit_bytes=48 * 1024 * 1024,
        ),
    )(xs, cw, cb, smat, fwp)

    o = out.reshape(_CLASSES, tiles, _BS, _BT)
    logits = jnp.stack([o[:, :, :, 0], o[:, :, :, _HW_IN]], axis=2)
    logits = logits.reshape(_CLASSES, n_pad)[:, :n].T
    return logits + fc_b.astype(jnp.float32)


# speedup vs baseline: 1.0176x; 1.0176x over previous
import jax
import jax.numpy as jnp
from jax import lax
from jax.experimental import pallas as pl
from jax.experimental.pallas import tpu as pltpu

_LAYERS = 8
_KW = 5
_HW_IN = 64
_HW_FC = 32
_CLASSES = 2
_SLOPE = 0.01
_BS = 64
_BT = 2 * _BS

_OUT_SIZES = [_HW_IN - layer * (_KW - 1) - (_KW - 1) for layer in range(_LAYERS)]
_MXU_ROWS = [(out * 3 + 2) // 5 for out in _OUT_SIZES]


def _convnet_body(x_ref, cw_ref, cb_ref, s_ref, fwp_ref, out_ref):
    half_a = x_ref[0:_BS, :, :]
    half_b = x_ref[_BS:2 * _BS, :, :]
    packed = jnp.concatenate([half_a, half_b], axis=2)
    act = jnp.transpose(packed, (1, 0, 2))
    size = _HW_IN
    for layer in range(_LAYERS):
        out_size = size - (_KW - 1)
        r = _MXU_ROWS[layer]
        pieces = []
        if r > 0:
            y = None
            for di in range(_KW):
                lhs = act[di:di + r].reshape(r * _BS, _BT)
                t = lax.dot_general(
                    lhs, s_ref[layer, di],
                    (((1,), (0,)), ((), ())),
                    preferred_element_type=jnp.float32,
                )
                y = t if y is None else y + t
            pieces.append(y.reshape(r, _BS, _BT))
        if r < out_size:
            tail = act[r:size]
            parts = []
            for dj in range(_KW):
                shv = tail if dj == 0 else pltpu.roll(tail, _BT - dj, 2)
                p = None
                for di in range(_KW):
                    w = cw_ref[layer * _KW * _KW + di * _KW + dj]
                    term = shv[di:di + out_size - r] * w
                    p = term if p is None else p + term
                parts.append(p)
            pieces.append(
                ((parts[0] + parts[1]) + (parts[2] + parts[3])) + parts[4])
        acc = pieces[0] if len(pieces) == 1 else jnp.concatenate(pieces, axis=0)
        acc = acc + cb_ref[layer]
        act = jnp.maximum(acc, _SLOPE * acc)
        size = out_size
    for c in range(_CLASSES):
        p = act * fwp_ref[c]
        s = jnp.sum(p, axis=0)
        for k in (32, 16, 8, 4, 2, 1):
            s = s + pltpu.roll(s, _BT - k, 1)
        out_ref[c, :, :] = s


def kernel(x, conv_w, conv_b, fc_w, fc_b):
    n = x.shape[0]
    n_pad = ((n + _BT - 1) // _BT) * _BT
    tiles = n_pad // _BT

    xs = x[:, 0, :, :].astype(jnp.float32)
    if n_pad != n:
        xs = jnp.pad(xs, ((0, n_pad - n), (0, 0), (0, 0)))

    cw = conv_w.reshape(_LAYERS * _KW * _KW).astype(jnp.float32)
    cb = conv_b.reshape(_LAYERS).astype(jnp.float32)
    ar = jnp.arange(_BT)
    kk, jj = ar[:, None], ar[None, :]
    same = (kk // _HW_IN) == (jj // _HW_IN)
    wl = conv_w.reshape(_LAYERS, _KW, _KW).astype(jnp.float32)
    smat = jnp.zeros((_LAYERS, _KW, _BT, _BT), jnp.float32)
    for dj in range(_KW):
        mask = ((kk - jj) == dj) & same
        smat = smat + jnp.where(mask, 1.0, 0.0) * wl[:, :, dj][:, :, None, None]
    fw = fc_w.reshape(_CLASSES, _HW_FC, _HW_FC).astype(jnp.float32)
    fw = jnp.pad(fw, ((0, 0), (0, 0), (0, _HW_IN - _HW_FC)))
    fwp = jnp.concatenate([fw, fw], axis=-1).reshape(_CLASSES, _HW_FC, 1, _BT)

    out = pl.pallas_call(
        _convnet_body,
        out_shape=jax.ShapeDtypeStruct((_CLASSES, n_pad // 2, _BT), jnp.float32),
        grid=(tiles,),
        in_specs=[
            pl.BlockSpec((_BT, _HW_IN, _HW_IN), lambda i: (i, 0, 0)),
            pl.BlockSpec(memory_space=pltpu.MemorySpace.SMEM),
            pl.BlockSpec(memory_space=pltpu.MemorySpace.SMEM),
            pl.BlockSpec(memory_space=pltpu.MemorySpace.VMEM),
            pl.BlockSpec(memory_space=pltpu.MemorySpace.VMEM),
        ],
        out_specs=pl.BlockSpec((_CLASSES, _BS, _BT), lambda i: (0, i, 0)),
        compiler_params=pltpu.CompilerParams(
            dimension_semantics=("parallel",),
            vmem_limit_bytes=48 * 1024 * 1024,
        ),
    )(xs, cw, cb, smat, fwp)

    o = out.reshape(_CLASSES, tiles, _BS, _BT)
    logits = jnp.stack([o[:, :, :, 0], o[:, :, :, _HW_IN]], axis=2)
    logits = logits.reshape(_CLASSES, n_pad)[:, :n].T
    return logits + fc_b.astype(jnp.float32)
